# baseline (speedup 1.0000x reference)
import functools
import jax
import jax.numpy as jnp
from jax import lax
from jax.experimental import pallas as pl

_NEG = -1e30
_NBC = 1000


def _cdiv(a, b):
    return (a + b - 1) // b


def _sseg_body(idx_ref, val_ref, out_ref, *, mode, post, n_eb, n_nc):
    eb = pl.program_id(0)

    @pl.when(eb == 0)
    def _():
        out_ref[...] = jnp.full(out_ref.shape, _NEG if mode == "max" else 0.0,
                                jnp.float32)

    idx = idx_ref[0, 0, :]
    val = val_ref[0, 0, :]
    for c in range(n_nc):
        ids = c * _NBC + lax.broadcasted_iota(jnp.int32,
                                              (idx.shape[0], _NBC), 1)
        oh = idx[:, None] == ids
        if mode == "max":
            contrib = jnp.max(jnp.where(oh, val[:, None], _NEG), axis=0)
            out_ref[c, :] = jnp.maximum(out_ref[c, :], contrib)
        else:
            contrib = jnp.sum(jnp.where(oh, val[:, None], 0.0), axis=0)
            out_ref[c, :] = out_ref[c, :] + contrib

    @pl.when(eb == n_eb - 1)
    def _():
        cur = out_ref[...]
        if post == "rsqrt":
            out_ref[...] = lax.rsqrt(cur)
        elif post == "gt0":
            out_ref[...] = jnp.where(cur > 0.0, 1.0, 0.0)


def _sseg(idx3, val3, n_nodes, mode="sum", post=None):
    n_eb, _, eb_sz = idx3.shape
    n_nc = _cdiv(n_nodes, _NBC)
    return pl.pallas_call(
        functools.partial(_sseg_body, mode=mode, post=post, n_eb=n_eb,
                          n_nc=n_nc),
        grid=(n_eb,),
        in_specs=[
            pl.BlockSpec((1, 1, eb_sz), lambda eb: (eb, 0, 0)),
            pl.BlockSpec((1, 1, eb_sz), lambda eb: (eb, 0, 0)),
        ],
        out_specs=pl.BlockSpec((n_nc, _NBC), lambda eb: (0, 0)),
        out_shape=jax.ShapeDtypeStruct((n_nc, _NBC), jnp.float32),
    )(idx3, val3)


def _gscal_body(idx_ref, tbl_ref, base_ref, out_ref, *, mode, post, n_nc):
    idx = idx_ref[0, 0, :]
    acc = jnp.full(idx.shape, _NEG if mode == "max" else 0.0, jnp.float32)
    for c in range(n_nc):
        ids = c * _NBC + lax.broadcasted_iota(jnp.int32,
                                              (idx.shape[0], _NBC), 1)
        oh = idx[:, None] == ids
        tbl = tbl_ref[c, :]
        if mode == "max":
            acc = jnp.maximum(acc, jnp.max(jnp.where(oh, tbl[None, :], _NEG),
                                           axis=1))
        else:
            acc = acc + jnp.sum(jnp.where(oh, tbl[None, :], 0.0), axis=1)
    b = base_ref[0, 0, :]
    if post == "add":
        out = b + acc
    elif post == "add_lrelu":
        s = b + acc
        out = jnp.where(s >= 0.0, s, 0.01 * s)
    elif post == "mul":
        out = b * acc
    elif post == "sub_exp":
        out = jnp.where(acc < -1e29, 0.0, jnp.exp(b - acc))
    elif post == "div":
        out = jnp.where(acc > 0.0, b / acc, 0.0)
    else:
        out = acc
    out_ref[0, 0, :] = out


def _gscal(tbl, idx3, base3, mode="sum", post="add"):
    n_eb, _, eb_sz = idx3.shape
    n_nc = tbl.shape[0]
    return pl.pallas_call(
        functools.partial(_gscal_body, mode=mode, post=post, n_nc=n_nc),
        grid=(n_eb,),
        in_specs=[
            pl.BlockSpec((1, 1, eb_sz), lambda eb: (eb, 0, 0)),
            pl.BlockSpec((n_nc, _NBC), lambda eb: (0, 0)),
            pl.BlockSpec((1, 1, eb_sz), lambda eb: (eb, 0, 0)),
        ],
        out_specs=pl.BlockSpec((1, 1, eb_sz), lambda eb: (eb, 0, 0)),
        out_shape=jax.ShapeDtypeStruct(idx3.shape, jnp.float32),
    )(idx3, tbl, base3)


def _grow_body(idx_ref, src_ref, scl_ref, out_ref, *, n_nc, nc_sz, scaled):
    eb, nc = pl.program_id(0), pl.program_id(1)

    @pl.when(nc == 0)
    def _():
        out_ref[...] = jnp.zeros(out_ref.shape, jnp.float32)

    idx = idx_ref[0, 0, :]
    ids = nc * nc_sz + lax.broadcasted_iota(jnp.int32, (idx.shape[0], nc_sz), 1)
    oh = (idx[:, None] == ids).astype(jnp.float32)
    out_ref[...] += lax.dot_general(
        oh, src_ref[...], (((1,), (0,)), ((), ())),
        preferred_element_type=jnp.float32)

    if scaled:
        @pl.when(nc == n_nc - 1)
        def _():
            out_ref[...] = out_ref[...] * scl_ref[0, 0, :][:, None]


def _grow(src, idx3, scale3, nc_sz):
    n_eb, _, eb_sz = idx3.shape
    n_nc = _cdiv(src.shape[0], nc_sz)
    f = src.shape[1]
    scaled = scale3 is not None
    if scale3 is None:
        scale3 = jnp.zeros(idx3.shape, jnp.float32)
    return pl.pallas_call(
        functools.partial(_grow_body, n_nc=n_nc, nc_sz=nc_sz, scaled=scaled),
        grid=(n_eb, n_nc),
        in_specs=[
            pl.BlockSpec((1, 1, eb_sz), lambda eb, nc: (eb, 0, 0)),
            pl.BlockSpec((nc_sz, f), lambda eb, nc: (nc, 0)),
            pl.BlockSpec((1, 1, eb_sz), lambda eb, nc: (eb, 0, 0)),
        ],
        out_specs=pl.BlockSpec((eb_sz, f), lambda eb, nc: (eb, 0)),
        out_shape=jax.ShapeDtypeStruct((n_eb * eb_sz, f), jnp.float32),
    )(idx3, src, scale3)


def _srow_body(idx_ref, msg_ref, out_ref, *, n_ec, nb_sz, relu):
    nb, ec = pl.program_id(0), pl.program_id(1)

    @pl.when(ec == 0)
    def _():
        out_ref[...] = jnp.zeros(out_ref.shape, jnp.float32)

    idx = idx_ref[0, 0, :]
    ids = nb * nb_sz + lax.broadcasted_iota(jnp.int32, (idx.shape[0], nb_sz), 1)
    oh = (idx[:, None] == ids).astype(jnp.float32)
    out_ref[...] += lax.dot_general(
        oh, msg_ref[...], (((0,), (0,)), ((), ())),
        preferred_element_type=jnp.float32)

    if relu:
        @pl.when(ec == n_ec - 1)
        def _():
            out_ref[...] = jnp.maximum(out_ref[...], 0.0)


def _srow(msg, idx3, n_nodes, nb_sz, relu=False):
    n_eb, _, eb_sz = idx3.shape
    f = msg.shape[1]
    n_nb = _cdiv(n_nodes, nb_sz)
    return pl.pallas_call(
        functools.partial(_srow_body, n_ec=n_eb, nb_sz=nb_sz, relu=relu),
        grid=(n_nb, n_eb),
        in_specs=[
            pl.BlockSpec((1, 1, eb_sz), lambda nb, ec: (ec, 0, 0)),
            pl.BlockSpec((eb_sz, f), lambda nb, ec: (ec, 0)),
        ],
        out_specs=pl.BlockSpec((nb_sz, f), lambda nb, ec: (nb, 0)),
        out_shape=jax.ShapeDtypeStruct((n_nodes, f), jnp.float32),
    )(idx3, msg)


def _dmm_body(x_ref, w_ref, out_ref, *, pre_relu):
    x = x_ref[...]
    if pre_relu:
        x = jnp.maximum(x, 0.0)
    out_ref[...] = lax.dot_general(x, w_ref[...], (((1,), (1,)), ((), ())),
                                   preferred_element_type=jnp.float32)


def _dmm(x, w, nb_sz, pre_relu=False):
    n, k = x.shape
    m = w.shape[0]
    return pl.pallas_call(
        functools.partial(_dmm_body, pre_relu=pre_relu),
        grid=(_cdiv(n, nb_sz),),
        in_specs=[
            pl.BlockSpec((nb_sz, k), lambda nb: (nb, 0)),
            pl.BlockSpec((m, k), lambda nb: (0, 0)),
        ],
        out_specs=pl.BlockSpec((nb_sz, m), lambda nb: (nb, 0)),
        out_shape=jax.ShapeDtypeStruct((n, m), jnp.float32),
    )(x, w)


def _hw_body(x1_ref, x2_ref, w_ref, b_ref, out_ref):
    x1 = x1_ref[...]
    g = jax.nn.sigmoid(
        lax.dot_general(x1, w_ref[...], (((1,), (1,)), ((), ())),
                        preferred_element_type=jnp.float32)
        + b_ref[0, :][None, :])
    out_ref[...] = g * x2_ref[...] + (1.0 - g) * x1


def _highway(x1, x2, w, b2, nb_sz):
    n, k = x1.shape
    return pl.pallas_call(
        _hw_body,
        grid=(_cdiv(n, nb_sz),),
        in_specs=[
            pl.BlockSpec((nb_sz, k), lambda nb: (nb, 0)),
            pl.BlockSpec((nb_sz, k), lambda nb: (nb, 0)),
            pl.BlockSpec((k, k), lambda nb: (0, 0)),
            pl.BlockSpec((1, k), lambda nb: (0, 0)),
        ],
        out_specs=pl.BlockSpec((nb_sz, k), lambda nb: (nb, 0)),
        out_shape=jax.ShapeDtypeStruct((n, k), jnp.float32),
    )(x1, x2, w, b2)


def _dmv_body(x_ref, w_ref, out_ref):
    nb = pl.program_id(0)
    out_ref[nb, :] = jnp.sum(x_ref[...] * w_ref[0, :][None, :], axis=1)


def _dmv(x, w):
    n, k = x.shape
    n_nc = _cdiv(n, _NBC)
    return pl.pallas_call(
        _dmv_body,
        grid=(n_nc,),
        in_specs=[
            pl.BlockSpec((_NBC, k), lambda nb: (nb, 0)),
            pl.BlockSpec((1, k), lambda nb: (0, 0)),
        ],
        out_specs=pl.BlockSpec((n_nc, _NBC), lambda nb: (0, 0)),
        out_shape=jax.ShapeDtypeStruct((n_nc, _NBC), jnp.float32),
    )(x, w.reshape(1, k))


def _rowscale_body(x_ref, s_ref, out_ref, *, relu):
    nb = pl.program_id(0)
    x = x_ref[...]
    if relu:
        x = jnp.maximum(x, 0.0)
    out_ref[...] = x * s_ref[nb, :][:, None]


def _rowscale(x, s, nb_sz, relu=False):
    n, k = x.shape
    n_nc = s.shape[0]
    return pl.pallas_call(
        functools.partial(_rowscale_body, relu=relu),
        grid=(_cdiv(n, nb_sz),),
        in_specs=[
            pl.BlockSpec((nb_sz, k), lambda nb: (nb, 0)),
            pl.BlockSpec((n_nc, _NBC), lambda nb: (0, 0)),
        ],
        out_specs=pl.BlockSpec((nb_sz, k), lambda nb: (nb, 0)),
        out_shape=jax.ShapeDtypeStruct((n, k), jnp.float32),
    )(x, s)


def _seg_softmax(e3, idx3, n_nodes):
    m = _sseg(idx3, e3, n_nodes, mode="max")
    ex3 = _gscal(m, idx3, e3, mode="max", post="sub_exp")
    s = _sseg(idx3, ex3, n_nodes, mode="sum")
    return _gscal(s, idx3, ex3, mode="sum", post="div")


def kernel(x_e, edge_index, rel, edge_index_all, rel_all,
           line_graph_index_out, line_graph_index_in, rel_emb1, rel_emb2,
           gcn1_W, gcn2_W, hw1_W, hw1_b, hw2_W, hw2_b, lgat_ai, lgat_aj, ww,
           gat_ai, gat_aj, gat_ar):
    N, EH = x_e.shape
    E = edge_index_all.shape[1]
    ELG = line_graph_index_out.shape[1]
    R = rel_emb1.shape[0]
    RH = rel_emb1.shape[1]

    EB, NB = 1600, 1000
    EBL = 3000

    r3 = lambda a, b: a.reshape(-1, 1, b)
    ei0 = r3(edge_index_all[0], EB)
    ei1 = r3(edge_index_all[1], EB)
    rall3 = r3(rel_all, EB)
    ones3 = jnp.ones((E // EB, 1, EB), jnp.float32)
    zeros3 = jnp.zeros((E // EB, 1, EB), jnp.float32)

    dis = _sseg(ei1, ones3, N, mode="sum", post="rsqrt")
    mask0 = _sseg(ei0, ones3, N, mode="sum", post="gt0")
    t = _gscal(dis, ei0, zeros3, post="add")
    norm3 = _gscal(dis, ei1, t, post="mul")

    msg = _grow(x_e, ei0, norm3, NB)
    agg = _srow(msg, ei1, N, NB, relu=True)
    x1 = _highway(x_e, _dmm(agg, gcn1_W, NB), hw1_W, hw1_b.reshape(1, EH), NB)

    msg = _grow(x1, ei0, norm3, NB)
    agg = _srow(msg, ei1, N, NB, relu=True)
    x2 = _highway(x1, _dmm(agg, gcn2_W, NB), hw2_W, hw2_b.reshape(1, EH), NB)

    use1 = (jnp.max(rel) + 1) == R
    base = jnp.where(use1, rel_emb1, rel_emb2)
    pb_i = _dmv(base, lgat_ai)
    pb_j = _dmv(base, lgat_aj)
    zl3 = jnp.zeros((ELG // EBL, 1, EBL), jnp.float32)

    def lgat(ei):
        lj = r3(ei[0], EBL)
        li = r3(ei[1], EBL)
        e = _gscal(pb_i, li, zl3, post="add")
        e = _gscal(pb_j, lj, e, post="add_lrelu")
        alpha = _seg_softmax(e, lj, R)
        m = _grow(base, lj, alpha, NB)
        return _srow(m, li, R, NB, relu=True)

    rel_emb = jnp.concatenate(
        [lgat(line_graph_index_out), lgat(line_graph_index_in)], axis=0)

    ww1, wwr, ww3 = ww[:EH], ww[EH:EH + RH], ww[EH + RH:]
    px1 = _dmv(x2, ww1)
    pr = _dmv(rel_emb, wwr)
    px3 = _dmv(x2, ww3)
    e = _gscal(px1, ei0, zeros3, post="add")
    e = _gscal(pr, rall3, e, post="add")
    e = _gscal(px3, ei1, e, post="add")
    att = _seg_softmax(e, ei0, N)
    c1 = _rowscale(x2, mask0, NB, relu=True)
    c2 = _srow(_grow(rel_emb, rall3, att, NB), ei0, N, NB, relu=True)
    c3 = _srow(_grow(x2, ei1, att, NB), ei0, N, NB, relu=True)
    x_rel = jnp.concatenate([x2, c1, c2, c3], axis=1)

    m_valid = edge_index_all[0] != edge_index_all[1]
    jf3 = r3(jnp.where(m_valid, edge_index_all[0], N), EB)
    if3 = r3(jnp.where(m_valid, edge_index_all[1], N), EB)
    pa_i = _dmv(x_rel, gat_ai)
    pa_j = _dmv(x_rel, gat_aj)
    par = _dmv(rel_emb, gat_ar)
    e = _gscal(pa_i, if3, zeros3, post="add")
    e = _gscal(pa_j, jf3, e, post="add")
    e = _gscal(par, rall3, e, post="add_lrelu")
    alpha = _seg_softmax(e, if3, N)
    g = _srow(_grow(x_rel, jf3, alpha, NB), if3, N, NB, relu=True)

    return jnp.concatenate([x_rel, g], axis=1)


# speedup vs baseline: 1.0961x; 1.0961x over previous
import functools
import jax
import jax.numpy as jnp
from jax import lax
from jax.experimental import pallas as pl

_NBC = 1000


def _cdiv(a, b):
    return (a + b - 1) // b


def _sseg_body(idx_ref, val_ref, out_ref, *, post, n_eb, n_nc):
    eb = pl.program_id(0)

    @pl.when(eb == 0)
    def _():
        out_ref[...] = jnp.zeros(out_ref.shape, jnp.float32)

    idx = idx_ref[0, 0, :]
    val2 = val_ref[0, 0, :][:, None]
    for c in range(n_nc):
        ids = c * _NBC + lax.broadcasted_iota(jnp.int32,
                                              (idx.shape[0], _NBC), 1)
        oh = (idx[:, None] == ids).astype(jnp.float32)
        contrib = lax.dot_general(oh, val2, (((0,), (0,)), ((), ())),
                                  preferred_element_type=jnp.float32)
        out_ref[c, :] = out_ref[c, :] + contrib[:, 0]

    @pl.when(eb == n_eb - 1)
    def _():
        if post == "rsqrt":
            out_ref[...] = lax.rsqrt(out_ref[...])


def _sseg(idx3, val3, n_nodes, post=None):
    n_eb, _, eb_sz = idx3.shape
    n_nc = _cdiv(n_nodes, _NBC)
    return pl.pallas_call(
        functools.partial(_sseg_body, post=post, n_eb=n_eb, n_nc=n_nc),
        grid=(n_eb,),
        in_specs=[
            pl.BlockSpec((1, 1, eb_sz), lambda eb: (eb, 0, 0)),
            pl.BlockSpec((1, 1, eb_sz), lambda eb: (eb, 0, 0)),
        ],
        out_specs=pl.BlockSpec((n_nc, _NBC), lambda eb: (0, 0)),
        out_shape=jax.ShapeDtypeStruct((n_nc, _NBC), jnp.float32),
    )(idx3, val3)


def _gscal_body(idx_ref, tbl_ref, base_ref, out_ref, *, post, n_nc, n_k,
                sentinel):
    idx = idx_ref[0, 0, :]
    acc = jnp.zeros((idx.shape[0], n_k), jnp.float32)
    for c in range(n_nc):
        ids = c * _NBC + lax.broadcasted_iota(jnp.int32,
                                              (idx.shape[0], _NBC), 1)
        oh = (idx[:, None] == ids).astype(jnp.float32)
        acc = acc + lax.dot_general(oh, tbl_ref[c], (((1,), (0,)), ((), ())),
                                    preferred_element_type=jnp.float32)
    b = base_ref[0, 0, :]
    if post == "raw2":
        for k in range(n_k):
            out_ref[0, k, :] = acc[:, k]
        return
    g = acc[:, 0]
    if post == "add":
        out = b + g
    elif post == "mul":
        out = b * g
    elif post == "add_exp":
        out = jnp.exp(b + g)
    elif post == "add_lrelu_exp":
        s = b + g
        out = jnp.exp(jnp.where(s >= 0.0, s, 0.01 * s))
    elif post == "add_lrelu_exp_mask":
        s = b + g
        out = jnp.where(idx == sentinel, 0.0,
                        jnp.exp(jnp.where(s >= 0.0, s, 0.01 * s)))
    elif post == "div":
        out = jnp.where(g > 0.0, b / g, 0.0)
    else:
        out = g
    out_ref[0, 0, :] = out


def _gscal(tbl, idx3, base3, post="add", n_k=1, sentinel=-1):
    n_eb, _, eb_sz = idx3.shape
    n_nc = tbl.shape[0]
    out_rows = n_k if post == "raw2" else 1
    return pl.pallas_call(
        functools.partial(_gscal_body, post=post, n_nc=n_nc, n_k=n_k,
                          sentinel=sentinel),
        grid=(n_eb,),
        in_specs=[
            pl.BlockSpec((1, 1, eb_sz), lambda eb: (eb, 0, 0)),
            pl.BlockSpec((n_nc, _NBC, n_k), lambda eb: (0, 0, 0)),
            pl.BlockSpec((1, 1, eb_sz), lambda eb: (eb, 0, 0)),
        ],
        out_specs=pl.BlockSpec((1, out_rows, eb_sz), lambda eb: (eb, 0, 0)),
        out_shape=jax.ShapeDtypeStruct((n_eb, out_rows, eb_sz), jnp.float32),
    )(idx3, tbl, base3)


def _grow_body(idx_ref, src_ref, scl_ref, out_ref, *, n_nc, nc_sz, scaled):
    eb, nc = pl.program_id(0), pl.program_id(1)

    @pl.when(nc == 0)
    def _():
        out_ref[...] = jnp.zeros(out_ref.shape, jnp.float32)

    idx = idx_ref[0, 0, :]
    ids = nc * nc_sz + lax.broadcasted_iota(jnp.int32, (idx.shape[0], nc_sz), 1)
    oh = (idx[:, None] == ids).astype(jnp.float32)
    out_ref[...] += lax.dot_general(
        oh, src_ref[...], (((1,), (0,)), ((), ())),
        preferred_element_type=jnp.float32)

    if scaled:
        @pl.when(nc == n_nc - 1)
        def _():
            out_ref[...] = out_ref[...] * scl_ref[0, 0, :][:, None]


def _grow(src, idx3, scale3, nc_sz):
    n_eb, _, eb_sz = idx3.shape
    n_nc = _cdiv(src.shape[0], nc_sz)
    f = src.shape[1]
    scaled = scale3 is not None
    if scale3 is None:
        scale3 = idx3
    return pl.pallas_call(
        functools.partial(_grow_body, n_nc=n_nc, nc_sz=nc_sz, scaled=scaled),
        grid=(n_eb, n_nc),
        in_specs=[
            pl.BlockSpec((1, 1, eb_sz), lambda eb, nc: (eb, 0, 0)),
            pl.BlockSpec((nc_sz, f), lambda eb, nc: (nc, 0)),
            pl.BlockSpec((1, 1, eb_sz), lambda eb, nc: (eb, 0, 0)),
        ],
        out_specs=pl.BlockSpec((eb_sz, f), lambda eb, nc: (eb, 0)),
        out_shape=jax.ShapeDtypeStruct((n_eb * eb_sz, f), jnp.float32),
    )(idx3, src, scale3)


def _srow_body(idx_ref, msg_ref, out_ref, *, n_ec, nb_sz, relu):
    nb, ec = pl.program_id(0), pl.program_id(1)

    @pl.when(ec == 0)
    def _():
        out_ref[...] = jnp.zeros(out_ref.shape, jnp.float32)

    idx = idx_ref[0, 0, :]
    ids = nb * nb_sz + lax.broadcasted_iota(jnp.int32, (idx.shape[0], nb_sz), 1)
    oh = (idx[:, None] == ids).astype(jnp.float32)
    out_ref[...] += lax.dot_general(
        oh, msg_ref[...], (((0,), (0,)), ((), ())),
        preferred_element_type=jnp.float32)

    if relu:
        @pl.when(ec == n_ec - 1)
        def _():
            out_ref[...] = jnp.maximum(out_ref[...], 0.0)


def _srow(msg, idx3, n_nodes, nb_sz, relu=False):
    n_eb, _, eb_sz = idx3.shape
    f = msg.shape[1]
    n_nb = _cdiv(n_nodes, nb_sz)
    return pl.pallas_call(
        functools.partial(_srow_body, n_ec=n_eb, nb_sz=nb_sz, relu=relu),
        grid=(n_nb, n_eb),
        in_specs=[
            pl.BlockSpec((1, 1, eb_sz), lambda nb, ec: (ec, 0, 0)),
            pl.BlockSpec((eb_sz, f), lambda nb, ec: (ec, 0)),
        ],
        out_specs=pl.BlockSpec((nb_sz, f), lambda nb, ec: (nb, 0)),
        out_shape=jax.ShapeDtypeStruct((n_nodes, f), jnp.float32),
    )(idx3, msg)


def _dmm_body(x_ref, w_ref, out_ref):
    out_ref[...] = lax.dot_general(x_ref[...], w_ref[...],
                                   (((1,), (1,)), ((), ())),
                                   preferred_element_type=jnp.float32)


def _dmm(x, w, nb_sz):
    n, k = x.shape
    m = w.shape[0]
    return pl.pallas_call(
        _dmm_body,
        grid=(_cdiv(n, nb_sz),),
        in_specs=[
            pl.BlockSpec((nb_sz, k), lambda nb: (nb, 0)),
            pl.BlockSpec((m, k), lambda nb: (0, 0)),
        ],
        out_specs=pl.BlockSpec((nb_sz, m), lambda nb: (nb, 0)),
        out_shape=jax.ShapeDtypeStruct((n, m), jnp.float32),
    )(x, w)


def _hw_body(x1_ref, x2_ref, w_ref, b_ref, out_ref):
    x1 = x1_ref[...]
    g = jax.nn.sigmoid(
        lax.dot_general(x1, w_ref[...], (((1,), (1,)), ((), ())),
                        preferred_element_type=jnp.float32)
        + b_ref[0, :][None, :])
    out_ref[...] = g * x2_ref[...] + (1.0 - g) * x1


def _highway(x1, x2, w, b2, nb_sz):
    n, k = x1.shape
    return pl.pallas_call(
        _hw_body,
        grid=(_cdiv(n, nb_sz),),
        in_specs=[
            pl.BlockSpec((nb_sz, k), lambda nb: (nb, 0)),
            pl.BlockSpec((nb_sz, k), lambda nb: (nb, 0)),
            pl.BlockSpec((k, k), lambda nb: (0, 0)),
            pl.BlockSpec((1, k), lambda nb: (0, 0)),
        ],
        out_specs=pl.BlockSpec((nb_sz, k), lambda nb: (nb, 0)),
        out_shape=jax.ShapeDtypeStruct((n, k), jnp.float32),
    )(x1, x2, w, b2)


def _dmv_body(x_ref, w_ref, out_ref, *, n_k):
    nb = pl.program_id(0)
    x = x_ref[...]
    for k in range(n_k):
        out_ref[nb, :, k] = jnp.sum(x * w_ref[k, :][None, :], axis=1)


def _dmv(x, ws):
    n, k = x.shape
    n_k = len(ws)
    w = jnp.stack([wi for wi in ws], axis=0)
    n_nc = _cdiv(n, _NBC)
    return pl.pallas_call(
        functools.partial(_dmv_body, n_k=n_k),
        grid=(n_nc,),
        in_specs=[
            pl.BlockSpec((_NBC, k), lambda nb: (nb, 0)),
            pl.BlockSpec((n_k, k), lambda nb: (0, 0)),
        ],
        out_specs=pl.BlockSpec((n_nc, _NBC, n_k), lambda nb: (0, 0, 0)),
        out_shape=jax.ShapeDtypeStruct((n_nc, _NBC, n_k), jnp.float32),
    )(x, w)


def _rowscale_body(x_ref, s_ref, out_ref, *, relu, gate):
    nb = pl.program_id(0)
    x = x_ref[...]
    if relu:
        x = jnp.maximum(x, 0.0)
    s = s_ref[nb, :, 0]
    if gate:
        s = jnp.where(s > 0.0, 1.0, 0.0)
    out_ref[...] = x * s[:, None]


def _rowscale(x, s, relu=False, gate=False):
    n, k = x.shape
    n_nc = s.shape[0]
    s3 = s if s.ndim == 3 else s[:, :, None]
    return pl.pallas_call(
        functools.partial(_rowscale_body, relu=relu, gate=gate),
        grid=(_cdiv(n, _NBC),),
        in_specs=[
            pl.BlockSpec((_NBC, k), lambda nb: (nb, 0)),
            pl.BlockSpec((n_nc, _NBC, 1), lambda nb: (0, 0, 0)),
        ],
        out_specs=pl.BlockSpec((_NBC, k), lambda nb: (nb, 0)),
        out_shape=jax.ShapeDtypeStruct((n, k), jnp.float32),
    )(x, s3)


def kernel(x_e, edge_index, rel, edge_index_all, rel_all,
           line_graph_index_out, line_graph_index_in, rel_emb1, rel_emb2,
           gcn1_W, gcn2_W, hw1_W, hw1_b, hw2_W, hw2_b, lgat_ai, lgat_aj, ww,
           gat_ai, gat_aj, gat_ar):
    N, EH = x_e.shape
    E = edge_index_all.shape[1]
    ELG = line_graph_index_out.shape[1]
    R = rel_emb1.shape[0]
    RH = rel_emb1.shape[1]

    EB, NB = 1600, 1000
    EBL = 3000

    r3 = lambda a, b: a.reshape(-1, 1, b)
    ei0 = r3(edge_index_all[0], EB)
    ei1 = r3(edge_index_all[1], EB)
    rall3 = r3(rel_all, EB)
    ones3 = jnp.ones((E // EB, 1, EB), jnp.float32)
    zeros3 = jnp.zeros((E // EB, 1, EB), jnp.float32)

    dis = _sseg(ei1, ones3, N, post="rsqrt")

    def gcn_highway(x, gw, hww, hwb):
        xs = _rowscale(x, dis)
        agg = _srow(_grow(xs, ei0, None, NB), ei1, N, NB)
        agg = _rowscale(agg, dis, relu=True)
        return _highway(x, _dmm(agg, gw, NB), hww, hwb.reshape(1, EH), NB)

    x1 = gcn_highway(x_e, gcn1_W, hw1_W, hw1_b)
    x2 = gcn_highway(x1, gcn2_W, hw2_W, hw2_b)

    use1 = (jnp.max(rel) + 1) == R
    base = jnp.where(use1, rel_emb1, rel_emb2)
    pb = _dmv(base, [lgat_ai, lgat_aj])
    pb_i, pb_j = pb[:, :, :1], pb[:, :, 1:]
    zl3 = jnp.zeros((ELG // EBL, 1, EBL), jnp.float32)

    def lgat(ei):
        lj = r3(ei[0], EBL)
        li = r3(ei[1], EBL)
        e = _gscal(pb_i, li, zl3, post="add")
        ee = _gscal(pb_j, lj, e, post="add_lrelu_exp")
        s = _sseg(lj, ee, R)
        alpha = _gscal(s[:, :, None], lj, ee, post="div")
        m = _grow(base, lj, alpha, NB)
        return _srow(m, li, R, NB, relu=True)

    rel_emb = jnp.concatenate(
        [lgat(line_graph_index_out), lgat(line_graph_index_in)], axis=0)

    ww1, wwr, ww3 = ww[:EH], ww[EH:EH + RH], ww[EH + RH:]
    px1 = _dmv(x2, [ww1])
    px3 = _dmv(x2, [ww3])
    prel = _dmv(rel_emb, [wwr, gat_ar])
    pg = _gscal(prel, rall3, zeros3, post="raw2", n_k=2)
    prg = pg[:, :1, :]
    parg = pg[:, 1:, :]
    e = _gscal(px1, ei0, prg, post="add")
    ee = _gscal(px3, ei1, e, post="add_exp")
    s_tw = _sseg(ei0, ee, N)
    att = _gscal(s_tw[:, :, None], ei0, ee, post="div")
    c1 = _rowscale(x2, s_tw, relu=True, gate=True)
    c2 = _srow(_grow(rel_emb, rall3, att, NB), ei0, N, NB, relu=True)
    c3 = _srow(_grow(x2, ei1, att, NB), ei0, N, NB, relu=True)
    x_rel = jnp.concatenate([x2, c1, c2, c3], axis=1)

    m_valid = edge_index_all[0] != edge_index_all[1]
    jf3 = r3(jnp.where(m_valid, edge_index_all[0], N), EB)
    if3 = r3(jnp.where(m_valid, edge_index_all[1], N), EB)
    pa_i = _dmv(x_rel, [gat_ai])
    pa_j = _dmv(x_rel, [gat_aj])
    e = _gscal(pa_i, if3, parg, post="add")
    ee = _gscal(pa_j, jf3, e, post="add_lrelu_exp_mask", sentinel=N)
    s_g = _sseg(if3, ee, N)
    alpha = _gscal(s_g[:, :, None], if3, ee, post="div")
    g = _srow(_grow(x_rel, jf3, alpha, NB), if3, N, NB, relu=True)

    return jnp.concatenate([x_rel, g], axis=1)
